# Initial kernel scaffold; baseline (speedup 1.0000x reference)
import jax
import jax.numpy as jnp
from jax.experimental import pallas as pl


def kernel(x_lr, y_lr, x_hr, w1, w2, w3, s1, b1, s2, b2):
    raise NotImplementedError("write your pallas kernel here")


# speedup vs baseline: 1.0663x; 1.0663x over previous
import numpy as np
import jax
import jax.numpy as jnp
from jax.experimental import pallas as pl
from jax.experimental.pallas import tpu as pltpu


def _box_w_matrix_t(n: int) -> np.ndarray:
    idx = np.arange(n)
    taps = (np.abs(idx[:, None] - idx[None, :]) <= 1).astype(np.float32)
    return np.ascontiguousarray((taps / taps.sum(axis=1, keepdims=True)).T)


def _resize_matrix(out_n: int, in_n: int) -> np.ndarray:
    if out_n == 1:
        m = np.zeros((1, in_n), np.float32)
        m[0, 0] = 1.0
        return m
    src = np.arange(out_n, dtype=np.float32) * (in_n - 1) / (out_n - 1)
    lo = np.clip(np.floor(src).astype(np.int32), 0, in_n - 1)
    hi = np.minimum(lo + 1, in_n - 1)
    t = src - lo.astype(np.float32)
    m = np.zeros((out_n, in_n), np.float32)
    np.add.at(m, (np.arange(out_n), lo), 1.0 - t)
    np.add.at(m, (np.arange(out_n), hi), t)
    return m


def _gf_kernel(x_lr_ref, y_lr_ref, swn_t_ref, uw_t_ref, uh_ref,
               w1_ref, s1_ref, b1_ref, w2_ref, s2_ref, b2_ref, w3_ref,
               x_hr_ref, out_ref, planes_ref):
    wh = out_ref.shape[3]
    j = pl.program_id(1)

    @pl.when(j == 0)
    def _coeffs():
        x = x_lr_ref[0]
        y = y_lr_ref[0]
        hl, wl = x.shape[1], x.shape[2]
        p12 = jnp.concatenate([x, y, x * y, x * x], axis=0).reshape(12 * hl, wl)

        qw = jnp.dot(p12, swn_t_ref[...], preferred_element_type=jnp.float32)
        r = jax.lax.broadcasted_iota(jnp.int32, qw.shape, 0) & (hl - 1)
        dn = pltpu.roll(qw, 1, axis=0)
        up = pltpu.roll(qw, 12 * hl - 1, axis=0)
        ssum = (qw + jnp.where(r == 0, 0.0, dn)
                + jnp.where(r == hl - 1, 0.0, up))
        nh = jnp.where(r == 0, 0.5, jnp.where(r == hl - 1, 0.5, 1.0 / 3.0))
        box = ssum * nh

        box3 = box.reshape(12, hl, wl)
        mx = box3[0:3]
        my = box3[3:6]
        cov = box3[6:9] - mx * my
        var = box3[9:12] - mx * mx

        feats = jnp.concatenate([cov, var], axis=0).reshape(6, hl * wl)
        h = jnp.dot(w1_ref[...], feats, preferred_element_type=jnp.float32)
        h = jnp.maximum(h * s1_ref[...] + b1_ref[...], 0.0)
        h = jnp.dot(w2_ref[...], h, preferred_element_type=jnp.float32)
        h = jnp.maximum(h * s2_ref[...] + b2_ref[...], 0.0)
        a = jnp.dot(w3_ref[...], h,
                    preferred_element_type=jnp.float32).reshape(3, hl, wl)
        bb = my - a * mx

        ab = jnp.concatenate([a, bb], axis=0).reshape(6 * hl, wl)
        m6 = jnp.dot(ab, uw_t_ref[...], preferred_element_type=jnp.float32)
        for c in range(3):
            planes_ref[:, (2 * c) * wh:(2 * c + 1) * wh] = m6[c * hl:(c + 1) * hl]
            planes_ref[:, (2 * c + 1) * wh:(2 * c + 2) * wh] = \
                m6[(3 + c) * hl:(4 + c) * hl]

    uh_blk = uh_ref[...]
    for c in range(3):
        pc = planes_ref[:, (2 * c) * wh:(2 * c + 2) * wh]
        m = jnp.dot(uh_blk, pc, preferred_element_type=jnp.float32)
        out_ref[0, c] = m[:, :wh] * x_hr_ref[0, c] + m[:, wh:]


def kernel(x_lr, y_lr, x_hr, w1, w2, w3, s1, b1, s2, b2):
    b, c, hl, wl = x_lr.shape
    _, _, hh, wh = x_hr.shape
    assert c == 3 and (hl & (hl - 1)) == 0

    swn_t = jnp.asarray(_box_w_matrix_t(wl))
    uw_t = jnp.asarray(_resize_matrix(wh, wl).T)
    uh = jnp.asarray(_resize_matrix(hh, hl))
    s1c, b1c = s1.reshape(-1, 1), b1.reshape(-1, 1)
    s2c, b2c = s2.reshape(-1, 1), b2.reshape(-1, 1)

    tile_h = 128 if hh % 128 == 0 else hh
    n_tiles = hh // tile_h

    cspec = lambda a: pl.BlockSpec(a.shape, lambda i, j: (0,) * a.ndim)

    flops = b * (2 * 12 * hl * wl * (hl + wl)
                 + 2 * hl * wl * (6 * 32 + 32 * 32 + 32 * 3)
                 + 2 * 6 * hl * wl * wh
                 + 2 * 6 * hh * hl * wh + 2 * 3 * hh * wh)
    bytes_accessed = b * 4 * (2 * 3 * hl * wl + 2 * 3 * hh * wh) + 4 * hh * hl

    return pl.pallas_call(
        _gf_kernel,
        out_shape=jax.ShapeDtypeStruct((b, 3, hh, wh), jnp.float32),
        grid=(b, n_tiles),
        in_specs=[
            pl.BlockSpec((1, 3, hl, wl), lambda i, j: (i, 0, 0, 0)),
            pl.BlockSpec((1, 3, hl, wl), lambda i, j: (i, 0, 0, 0)),
            cspec(swn_t), cspec(uw_t),
            pl.BlockSpec((tile_h, hl), lambda i, j: (j, 0)),
            cspec(w1), cspec(s1c), cspec(b1c),
            cspec(w2), cspec(s2c), cspec(b2c),
            cspec(w3),
            pl.BlockSpec((1, 3, tile_h, wh), lambda i, j: (i, 0, j, 0)),
        ],
        out_specs=pl.BlockSpec((1, 3, tile_h, wh), lambda i, j: (i, 0, j, 0)),
        scratch_shapes=[pltpu.VMEM((hl, 6 * wh), jnp.float32)],
        compiler_params=pltpu.CompilerParams(
            dimension_semantics=("parallel", "arbitrary"),
            vmem_limit_bytes=48 * 1024 * 1024,
        ),
        cost_estimate=pl.CostEstimate(flops=flops, transcendentals=0,
                                      bytes_accessed=bytes_accessed),
    )(x_lr, y_lr, swn_t, uw_t, uh, w1, s1c, b1c, w2, s2c, b2c, w3, x_hr)


# speedup vs baseline: 1.4024x; 1.3153x over previous
import numpy as np
import jax
import jax.numpy as jnp
from jax.experimental import pallas as pl
from jax.experimental.pallas import tpu as pltpu


def _box_w_matrix_t(n: int) -> np.ndarray:
    idx = np.arange(n)
    taps = (np.abs(idx[:, None] - idx[None, :]) <= 1).astype(np.float32)
    return np.ascontiguousarray((taps / taps.sum(axis=1, keepdims=True)).T)


def _resize_matrix(out_n: int, in_n: int) -> np.ndarray:
    if out_n == 1:
        m = np.zeros((1, in_n), np.float32)
        m[0, 0] = 1.0
        return m
    src = np.arange(out_n, dtype=np.float32) * (in_n - 1) / (out_n - 1)
    lo = np.clip(np.floor(src).astype(np.int32), 0, in_n - 1)
    hi = np.minimum(lo + 1, in_n - 1)
    t = src - lo.astype(np.float32)
    m = np.zeros((out_n, in_n), np.float32)
    np.add.at(m, (np.arange(out_n), lo), 1.0 - t)
    np.add.at(m, (np.arange(out_n), hi), t)
    return m


def _gf_kernel(x_lr_ref, y_lr_ref, swn_t_ref, uw_t_ref, uh_ref,
               w1_ref, s1_ref, b1_ref, w2_ref, s2_ref, b2_ref, w3_ref,
               x_hr_ref, out_ref, planes_ref):
    wh = out_ref.shape[3]
    j = pl.program_id(1)

    @pl.when(j == 0)
    def _coeffs():
        x = x_lr_ref[0]
        y = y_lr_ref[0]
        hl, wl = x.shape[1], x.shape[2]
        p12 = jnp.concatenate([x, y, x * y, x * x], axis=0).reshape(12 * hl, wl)

        qw = jnp.dot(p12, swn_t_ref[...], preferred_element_type=jnp.float32)
        r = jax.lax.broadcasted_iota(jnp.int32, qw.shape, 0) & (hl - 1)
        dn = pltpu.roll(qw, 1, axis=0)
        up = pltpu.roll(qw, 12 * hl - 1, axis=0)
        ssum = (qw + jnp.where(r == 0, 0.0, dn)
                + jnp.where(r == hl - 1, 0.0, up))
        nh = jnp.where(r == 0, 0.5, jnp.where(r == hl - 1, 0.5, 1.0 / 3.0))
        box = ssum * nh

        box3 = box.reshape(12, hl, wl)
        mx = box3[0:3]
        my = box3[3:6]
        cov = box3[6:9] - mx * my
        var = box3[9:12] - mx * mx

        feats = jnp.concatenate([cov, var], axis=0).reshape(6, hl * wl)
        h = jnp.dot(w1_ref[...], feats, preferred_element_type=jnp.float32)
        h = jnp.maximum(h * s1_ref[...] + b1_ref[...], 0.0)
        h = jnp.dot(w2_ref[...], h, preferred_element_type=jnp.float32)
        h = jnp.maximum(h * s2_ref[...] + b2_ref[...], 0.0)
        a = jnp.dot(w3_ref[...], h,
                    preferred_element_type=jnp.float32).reshape(3, hl, wl)
        bb = my - a * mx

        ab = jnp.concatenate([a, bb], axis=0).reshape(6 * hl, wl)
        m6 = jnp.dot(ab, uw_t_ref[...], preferred_element_type=jnp.float32)
        for c in range(3):
            planes_ref[:, (2 * c) * wh:(2 * c + 1) * wh] = m6[c * hl:(c + 1) * hl]
            planes_ref[:, (2 * c + 1) * wh:(2 * c + 2) * wh] = \
                m6[(3 + c) * hl:(4 + c) * hl]

    uh_blk = uh_ref[...]
    for c in range(3):
        pc = planes_ref[:, (2 * c) * wh:(2 * c + 2) * wh]
        m = jnp.dot(uh_blk, pc, preferred_element_type=jnp.float32)
        out_ref[0, c] = m[:, :wh] * x_hr_ref[0, c] + m[:, wh:]


def kernel(x_lr, y_lr, x_hr, w1, w2, w3, s1, b1, s2, b2):
    b, c, hl, wl = x_lr.shape
    _, _, hh, wh = x_hr.shape
    assert c == 3 and (hl & (hl - 1)) == 0

    swn_t = jnp.asarray(_box_w_matrix_t(wl))
    uw_t = jnp.asarray(_resize_matrix(wh, wl).T)
    uh = jnp.asarray(_resize_matrix(hh, hl))
    s1c, b1c = s1.reshape(-1, 1), b1.reshape(-1, 1)
    s2c, b2c = s2.reshape(-1, 1), b2.reshape(-1, 1)

    tile_h = 256 if hh % 256 == 0 else (128 if hh % 128 == 0 else hh)
    n_tiles = hh // tile_h

    cspec = lambda a: pl.BlockSpec(a.shape, lambda i, j: (0,) * a.ndim)

    flops = b * (2 * 12 * hl * wl * (hl + wl)
                 + 2 * hl * wl * (6 * 32 + 32 * 32 + 32 * 3)
                 + 2 * 6 * hl * wl * wh
                 + 2 * 6 * hh * hl * wh + 2 * 3 * hh * wh)
    bytes_accessed = b * 4 * (2 * 3 * hl * wl + 2 * 3 * hh * wh) + 4 * hh * hl

    return pl.pallas_call(
        _gf_kernel,
        out_shape=jax.ShapeDtypeStruct((b, 3, hh, wh), jnp.float32),
        grid=(b, n_tiles),
        in_specs=[
            pl.BlockSpec((1, 3, hl, wl), lambda i, j: (i, 0, 0, 0)),
            pl.BlockSpec((1, 3, hl, wl), lambda i, j: (i, 0, 0, 0)),
            cspec(swn_t), cspec(uw_t),
            pl.BlockSpec((tile_h, hl), lambda i, j: (j, 0)),
            cspec(w1), cspec(s1c), cspec(b1c),
            cspec(w2), cspec(s2c), cspec(b2c),
            cspec(w3),
            pl.BlockSpec((1, 3, tile_h, wh), lambda i, j: (i, 0, j, 0)),
        ],
        out_specs=pl.BlockSpec((1, 3, tile_h, wh), lambda i, j: (i, 0, j, 0)),
        scratch_shapes=[pltpu.VMEM((hl, 6 * wh), jnp.float32)],
        compiler_params=pltpu.CompilerParams(
            dimension_semantics=("parallel", "arbitrary"),
            vmem_limit_bytes=48 * 1024 * 1024,
        ),
        cost_estimate=pl.CostEstimate(flops=flops, transcendentals=0,
                                      bytes_accessed=bytes_accessed),
    )(x_lr, y_lr, swn_t, uw_t, uh, w1, s1c, b1c, w2, s2c, b2c, w3, x_hr)


# speedup vs baseline: 1.6902x; 1.2052x over previous
import numpy as np
import jax
import jax.numpy as jnp
from jax.experimental import pallas as pl
from jax.experimental.pallas import tpu as pltpu


def _box_w_matrix_t(n: int) -> np.ndarray:
    idx = np.arange(n)
    taps = (np.abs(idx[:, None] - idx[None, :]) <= 1).astype(np.float32)
    return np.ascontiguousarray((taps / taps.sum(axis=1, keepdims=True)).T)


def _resize_matrix(out_n: int, in_n: int) -> np.ndarray:
    if out_n == 1:
        m = np.zeros((1, in_n), np.float32)
        m[0, 0] = 1.0
        return m
    src = np.arange(out_n, dtype=np.float32) * (in_n - 1) / (out_n - 1)
    lo = np.clip(np.floor(src).astype(np.int32), 0, in_n - 1)
    hi = np.minimum(lo + 1, in_n - 1)
    t = src - lo.astype(np.float32)
    m = np.zeros((out_n, in_n), np.float32)
    np.add.at(m, (np.arange(out_n), lo), 1.0 - t)
    np.add.at(m, (np.arange(out_n), hi), t)
    return m


def _gf_kernel(x_lr_ref, y_lr_ref, swn_t_ref, uw_t_ref, uh_ref,
               w1_ref, s1_ref, b1_ref, w2_ref, s2_ref, b2_ref, w3_ref,
               x_hr_ref, out_ref, planes_ref):
    wh = out_ref.shape[3]
    j = pl.program_id(1)

    @pl.when(j == 0)
    def _coeffs():
        x = x_lr_ref[0]
        y = y_lr_ref[0]
        hl, wl = x.shape[1], x.shape[2]
        p12 = jnp.concatenate([x, y, x * y, x * x], axis=0).reshape(12 * hl, wl)

        qw = jnp.dot(p12, swn_t_ref[...], preferred_element_type=jnp.float32)
        r = jax.lax.broadcasted_iota(jnp.int32, qw.shape, 0) & (hl - 1)
        dn = pltpu.roll(qw, 1, axis=0)
        up = pltpu.roll(qw, 12 * hl - 1, axis=0)
        ssum = (qw + jnp.where(r == 0, 0.0, dn)
                + jnp.where(r == hl - 1, 0.0, up))
        nh = jnp.where(r == 0, 0.5, jnp.where(r == hl - 1, 0.5, 1.0 / 3.0))
        box = ssum * nh

        box3 = box.reshape(12, hl, wl)
        mx = box3[0:3]
        my = box3[3:6]
        cov = box3[6:9] - mx * my
        var = box3[9:12] - mx * mx

        feats = jnp.concatenate([cov, var], axis=0).reshape(6, hl * wl)
        h = jnp.dot(w1_ref[...], feats, preferred_element_type=jnp.float32)
        h = jnp.maximum(h * s1_ref[...] + b1_ref[...], 0.0)
        h = jnp.dot(w2_ref[...], h, preferred_element_type=jnp.float32)
        h = jnp.maximum(h * s2_ref[...] + b2_ref[...], 0.0)
        a = jnp.dot(w3_ref[...], h,
                    preferred_element_type=jnp.float32).reshape(3, hl, wl)
        bb = my - a * mx

        ab = jnp.concatenate([a, bb], axis=0).reshape(6 * hl, wl)
        m6 = jnp.dot(ab, uw_t_ref[...], preferred_element_type=jnp.float32)
        for c in range(3):
            planes_ref[:, (2 * c) * wh:(2 * c + 1) * wh] = m6[c * hl:(c + 1) * hl]
            planes_ref[:, (2 * c + 1) * wh:(2 * c + 2) * wh] = \
                m6[(3 + c) * hl:(4 + c) * hl]

    uh_blk = uh_ref[...]
    for c in range(3):
        pc = planes_ref[:, (2 * c) * wh:(2 * c + 2) * wh]
        m = jnp.dot(uh_blk, pc, preferred_element_type=jnp.float32)
        out_ref[0, c] = m[:, :wh] * x_hr_ref[0, c] + m[:, wh:]


def kernel(x_lr, y_lr, x_hr, w1, w2, w3, s1, b1, s2, b2):
    b, c, hl, wl = x_lr.shape
    _, _, hh, wh = x_hr.shape
    assert c == 3 and (hl & (hl - 1)) == 0

    swn_t = jnp.asarray(_box_w_matrix_t(wl))
    uw_t = jnp.asarray(_resize_matrix(wh, wl).T)
    uh = jnp.asarray(_resize_matrix(hh, hl))
    s1c, b1c = s1.reshape(-1, 1), b1.reshape(-1, 1)
    s2c, b2c = s2.reshape(-1, 1), b2.reshape(-1, 1)

    tile_h = 512 if hh % 512 == 0 else (128 if hh % 128 == 0 else hh)
    n_tiles = hh // tile_h

    cspec = lambda a: pl.BlockSpec(a.shape, lambda i, j: (0,) * a.ndim)

    flops = b * (2 * 12 * hl * wl * (hl + wl)
                 + 2 * hl * wl * (6 * 32 + 32 * 32 + 32 * 3)
                 + 2 * 6 * hl * wl * wh
                 + 2 * 6 * hh * hl * wh + 2 * 3 * hh * wh)
    bytes_accessed = b * 4 * (2 * 3 * hl * wl + 2 * 3 * hh * wh) + 4 * hh * hl

    return pl.pallas_call(
        _gf_kernel,
        out_shape=jax.ShapeDtypeStruct((b, 3, hh, wh), jnp.float32),
        grid=(b, n_tiles),
        in_specs=[
            pl.BlockSpec((1, 3, hl, wl), lambda i, j: (i, 0, 0, 0)),
            pl.BlockSpec((1, 3, hl, wl), lambda i, j: (i, 0, 0, 0)),
            cspec(swn_t), cspec(uw_t),
            pl.BlockSpec((tile_h, hl), lambda i, j: (j, 0)),
            cspec(w1), cspec(s1c), cspec(b1c),
            cspec(w2), cspec(s2c), cspec(b2c),
            cspec(w3),
            pl.BlockSpec((1, 3, tile_h, wh), lambda i, j: (i, 0, j, 0)),
        ],
        out_specs=pl.BlockSpec((1, 3, tile_h, wh), lambda i, j: (i, 0, j, 0)),
        scratch_shapes=[pltpu.VMEM((hl, 6 * wh), jnp.float32)],
        compiler_params=pltpu.CompilerParams(
            dimension_semantics=("parallel", "arbitrary"),
            vmem_limit_bytes=48 * 1024 * 1024,
        ),
        cost_estimate=pl.CostEstimate(flops=flops, transcendentals=0,
                                      bytes_accessed=bytes_accessed),
    )(x_lr, y_lr, swn_t, uw_t, uh, w1, s1c, b1c, w2, s2c, b2c, w3, x_hr)


# speedup vs baseline: 1.9777x; 1.1701x over previous
import numpy as np
import jax
import jax.numpy as jnp
from jax.experimental import pallas as pl
from jax.experimental.pallas import tpu as pltpu


def _box_w_matrix_t(n: int) -> np.ndarray:
    idx = np.arange(n)
    taps = (np.abs(idx[:, None] - idx[None, :]) <= 1).astype(np.float32)
    return np.ascontiguousarray((taps / taps.sum(axis=1, keepdims=True)).T)


def _resize_matrix(out_n: int, in_n: int) -> np.ndarray:
    if out_n == 1:
        m = np.zeros((1, in_n), np.float32)
        m[0, 0] = 1.0
        return m
    src = np.arange(out_n, dtype=np.float32) * (in_n - 1) / (out_n - 1)
    lo = np.clip(np.floor(src).astype(np.int32), 0, in_n - 1)
    hi = np.minimum(lo + 1, in_n - 1)
    t = src - lo.astype(np.float32)
    m = np.zeros((out_n, in_n), np.float32)
    np.add.at(m, (np.arange(out_n), lo), 1.0 - t)
    np.add.at(m, (np.arange(out_n), hi), t)
    return m


def _gf_kernel(x_lr_ref, y_lr_ref, swn_t_ref, uw_t_ref, uh_ref,
               w1_ref, s1_ref, b1_ref, w2_ref, s2_ref, b2_ref, w3_ref,
               x_hr_ref, out_ref, planes_ref):
    wh = out_ref.shape[3]
    j = pl.program_id(1)

    @pl.when(j == 0)
    def _coeffs():
        x = x_lr_ref[0]
        y = y_lr_ref[0]
        hl, wl = x.shape[1], x.shape[2]
        p12 = jnp.concatenate([x, y, x * y, x * x], axis=0).reshape(12 * hl, wl)

        qw = jnp.dot(p12, swn_t_ref[...], preferred_element_type=jnp.float32)
        r = jax.lax.broadcasted_iota(jnp.int32, qw.shape, 0) & (hl - 1)
        dn = pltpu.roll(qw, 1, axis=0)
        up = pltpu.roll(qw, 12 * hl - 1, axis=0)
        ssum = (qw + jnp.where(r == 0, 0.0, dn)
                + jnp.where(r == hl - 1, 0.0, up))
        nh = jnp.where(r == 0, 0.5, jnp.where(r == hl - 1, 0.5, 1.0 / 3.0))
        box = ssum * nh

        box3 = box.reshape(12, hl, wl)
        mx = box3[0:3]
        my = box3[3:6]
        cov = box3[6:9] - mx * my
        var = box3[9:12] - mx * mx

        feats = jnp.concatenate([cov, var], axis=0).reshape(6, hl * wl)
        h = jnp.dot(w1_ref[...], feats, preferred_element_type=jnp.float32)
        h = jnp.maximum(h * s1_ref[...] + b1_ref[...], 0.0)
        h = jnp.dot(w2_ref[...], h, preferred_element_type=jnp.float32)
        h = jnp.maximum(h * s2_ref[...] + b2_ref[...], 0.0)
        a = jnp.dot(w3_ref[...], h,
                    preferred_element_type=jnp.float32).reshape(3, hl, wl)
        bb = my - a * mx

        ab = jnp.concatenate([a, bb], axis=0).reshape(6 * hl, wl)
        m6 = jnp.dot(ab, uw_t_ref[...], preferred_element_type=jnp.float32)
        for c in range(3):
            planes_ref[:, (2 * c) * wh:(2 * c + 1) * wh] = m6[c * hl:(c + 1) * hl]
            planes_ref[:, (2 * c + 1) * wh:(2 * c + 2) * wh] = \
                m6[(3 + c) * hl:(4 + c) * hl]

    uh_blk = uh_ref[...]
    for c in range(3):
        pc = planes_ref[:, (2 * c) * wh:(2 * c + 2) * wh]
        m = jnp.dot(uh_blk, pc, preferred_element_type=jnp.float32)
        out_ref[0, c] = m[:, :wh] * x_hr_ref[0, c] + m[:, wh:]


def kernel(x_lr, y_lr, x_hr, w1, w2, w3, s1, b1, s2, b2):
    b, c, hl, wl = x_lr.shape
    _, _, hh, wh = x_hr.shape
    assert c == 3 and (hl & (hl - 1)) == 0

    swn_t = jnp.asarray(_box_w_matrix_t(wl))
    uw_t = jnp.asarray(_resize_matrix(wh, wl).T)
    uh = jnp.asarray(_resize_matrix(hh, hl))
    s1c, b1c = s1.reshape(-1, 1), b1.reshape(-1, 1)
    s2c, b2c = s2.reshape(-1, 1), b2.reshape(-1, 1)

    tile_h = hh if hh <= 1024 else (512 if hh % 512 == 0 else hh)
    n_tiles = hh // tile_h

    cspec = lambda a: pl.BlockSpec(a.shape, lambda i, j: (0,) * a.ndim)

    flops = b * (2 * 12 * hl * wl * (hl + wl)
                 + 2 * hl * wl * (6 * 32 + 32 * 32 + 32 * 3)
                 + 2 * 6 * hl * wl * wh
                 + 2 * 6 * hh * hl * wh + 2 * 3 * hh * wh)
    bytes_accessed = b * 4 * (2 * 3 * hl * wl + 2 * 3 * hh * wh) + 4 * hh * hl

    return pl.pallas_call(
        _gf_kernel,
        out_shape=jax.ShapeDtypeStruct((b, 3, hh, wh), jnp.float32),
        grid=(b, n_tiles),
        in_specs=[
            pl.BlockSpec((1, 3, hl, wl), lambda i, j: (i, 0, 0, 0)),
            pl.BlockSpec((1, 3, hl, wl), lambda i, j: (i, 0, 0, 0)),
            cspec(swn_t), cspec(uw_t),
            pl.BlockSpec((tile_h, hl), lambda i, j: (j, 0)),
            cspec(w1), cspec(s1c), cspec(b1c),
            cspec(w2), cspec(s2c), cspec(b2c),
            cspec(w3),
            pl.BlockSpec((1, 3, tile_h, wh), lambda i, j: (i, 0, j, 0)),
        ],
        out_specs=pl.BlockSpec((1, 3, tile_h, wh), lambda i, j: (i, 0, j, 0)),
        scratch_shapes=[pltpu.VMEM((hl, 6 * wh), jnp.float32)],
        compiler_params=pltpu.CompilerParams(
            dimension_semantics=("parallel", "arbitrary"),
            vmem_limit_bytes=48 * 1024 * 1024,
        ),
        cost_estimate=pl.CostEstimate(flops=flops, transcendentals=0,
                                      bytes_accessed=bytes_accessed),
    )(x_lr, y_lr, swn_t, uw_t, uh, w1, s1c, b1c, w2, s2c, b2c, w3, x_hr)


# speedup vs baseline: 2.0472x; 1.0351x over previous
import numpy as np
import jax
import jax.numpy as jnp
from jax.experimental import pallas as pl
from jax.experimental.pallas import tpu as pltpu


def _box_w_matrix_t(n: int) -> np.ndarray:
    idx = np.arange(n)
    taps = (np.abs(idx[:, None] - idx[None, :]) <= 1).astype(np.float32)
    return np.ascontiguousarray((taps / taps.sum(axis=1, keepdims=True)).T)


def _resize_matrix(out_n: int, in_n: int) -> np.ndarray:
    if out_n == 1:
        m = np.zeros((1, in_n), np.float32)
        m[0, 0] = 1.0
        return m
    src = np.arange(out_n, dtype=np.float32) * (in_n - 1) / (out_n - 1)
    lo = np.clip(np.floor(src).astype(np.int32), 0, in_n - 1)
    hi = np.minimum(lo + 1, in_n - 1)
    t = src - lo.astype(np.float32)
    m = np.zeros((out_n, in_n), np.float32)
    np.add.at(m, (np.arange(out_n), lo), 1.0 - t)
    np.add.at(m, (np.arange(out_n), hi), t)
    return m


def _gf_kernel(x_lr_ref, y_lr_ref, swn_t_ref, uw_t_ref, uh_ref,
               w1_ref, s1_ref, b1_ref, w2_ref, s2_ref, b2_ref, w3_ref,
               x_hr_ref, out_ref, planes_ref):
    nb = out_ref.shape[0]
    wh = out_ref.shape[3]
    uh_blk = uh_ref[...]

    for k in range(nb):
        x = x_lr_ref[k]
        y = y_lr_ref[k]
        hl, wl = x.shape[1], x.shape[2]
        p12 = jnp.concatenate([x, y, x * y, x * x], axis=0).reshape(12 * hl, wl)

        qw = jnp.dot(p12, swn_t_ref[...], preferred_element_type=jnp.float32)
        r = jax.lax.broadcasted_iota(jnp.int32, qw.shape, 0) & (hl - 1)
        dn = pltpu.roll(qw, 1, axis=0)
        up = pltpu.roll(qw, 12 * hl - 1, axis=0)
        ssum = (qw + jnp.where(r == 0, 0.0, dn)
                + jnp.where(r == hl - 1, 0.0, up))
        nh = jnp.where(r == 0, 0.5, jnp.where(r == hl - 1, 0.5, 1.0 / 3.0))
        box = ssum * nh

        box3 = box.reshape(12, hl, wl)
        mx = box3[0:3]
        my = box3[3:6]
        cov = box3[6:9] - mx * my
        var = box3[9:12] - mx * mx

        feats = jnp.concatenate([cov, var], axis=0).reshape(6, hl * wl)
        h = jnp.dot(w1_ref[...], feats, preferred_element_type=jnp.float32)
        h = jnp.maximum(h * s1_ref[...] + b1_ref[...], 0.0)
        h = jnp.dot(w2_ref[...], h, preferred_element_type=jnp.float32)
        h = jnp.maximum(h * s2_ref[...] + b2_ref[...], 0.0)
        a = jnp.dot(w3_ref[...], h,
                    preferred_element_type=jnp.float32).reshape(3, hl, wl)
        bb = my - a * mx

        ab = jnp.concatenate([a, bb], axis=0).reshape(6 * hl, wl)
        m6 = jnp.dot(ab, uw_t_ref[...], preferred_element_type=jnp.float32)
        for c in range(3):
            planes_ref[:, (2 * c) * wh:(2 * c + 1) * wh] = m6[c * hl:(c + 1) * hl]
            planes_ref[:, (2 * c + 1) * wh:(2 * c + 2) * wh] = \
                m6[(3 + c) * hl:(4 + c) * hl]

        for c in range(3):
            pc = planes_ref[:, (2 * c) * wh:(2 * c + 2) * wh]
            m = jnp.dot(uh_blk, pc, preferred_element_type=jnp.float32)
            out_ref[k, c] = m[:, :wh] * x_hr_ref[k, c] + m[:, wh:]


def kernel(x_lr, y_lr, x_hr, w1, w2, w3, s1, b1, s2, b2):
    b, c, hl, wl = x_lr.shape
    _, _, hh, wh = x_hr.shape
    assert c == 3 and (hl & (hl - 1)) == 0

    swn_t = jnp.asarray(_box_w_matrix_t(wl))
    uw_t = jnp.asarray(_resize_matrix(wh, wl).T)
    uh = jnp.asarray(_resize_matrix(hh, hl))
    s1c, b1c = s1.reshape(-1, 1), b1.reshape(-1, 1)
    s2c, b2c = s2.reshape(-1, 1), b2.reshape(-1, 1)

    nb = 2 if b % 2 == 0 else 1
    grid = (b // nb,)

    cspec = lambda a: pl.BlockSpec(a.shape, lambda i: (0,) * a.ndim)

    flops = b * (2 * 12 * hl * wl * (hl + wl)
                 + 2 * hl * wl * (6 * 32 + 32 * 32 + 32 * 3)
                 + 2 * 6 * hl * wl * wh
                 + 2 * 6 * hh * hl * wh + 2 * 3 * hh * wh)
    bytes_accessed = b * 4 * (2 * 3 * hl * wl + 2 * 3 * hh * wh) + 4 * hh * hl

    return pl.pallas_call(
        _gf_kernel,
        out_shape=jax.ShapeDtypeStruct((b, 3, hh, wh), jnp.float32),
        grid=grid,
        in_specs=[
            pl.BlockSpec((nb, 3, hl, wl), lambda i: (i, 0, 0, 0)),
            pl.BlockSpec((nb, 3, hl, wl), lambda i: (i, 0, 0, 0)),
            cspec(swn_t), cspec(uw_t), cspec(uh),
            cspec(w1), cspec(s1c), cspec(b1c),
            cspec(w2), cspec(s2c), cspec(b2c),
            cspec(w3),
            pl.BlockSpec((nb, 3, hh, wh), lambda i: (i, 0, 0, 0)),
        ],
        out_specs=pl.BlockSpec((nb, 3, hh, wh), lambda i: (i, 0, 0, 0)),
        scratch_shapes=[pltpu.VMEM((hl, 6 * wh), jnp.float32)],
        compiler_params=pltpu.CompilerParams(
            dimension_semantics=("parallel",),
            vmem_limit_bytes=57 * 1024 * 1024,
        ),
        cost_estimate=pl.CostEstimate(flops=flops, transcendentals=0,
                                      bytes_accessed=bytes_accessed),
    )(x_lr, y_lr, swn_t, uw_t, uh, w1, s1c, b1c, w2, s2c, b2c, w3, x_hr)
